# baseline (speedup 1.0000x reference)
import functools

import jax
import jax.numpy as jnp
from jax import lax
from jax.experimental import pallas as pl
from jax.experimental.pallas import tpu as pltpu
from jax.experimental.pallas import tpu_sc as plsc

N_ATOM = 10000
N_RES = 1000
N_RES_PAD = 1024
E_BOND = 20000
E_ATOM = 160000
E_RES = 32000
D = 128

NC = 2
NS = 16
DH = D // NC

F32 = jnp.float32


def _make_edge_agg(n_nodes: int, n_edges: int, k_chunk: int):
    assert n_edges % (NS * k_chunk) == 0 and k_chunk % 8 == 0
    assert n_nodes % NS == 0
    et = n_edges // NS
    n_chunks = et // k_chunk
    rt = n_nodes // NS
    zpieces = []
    off = 0
    while off < rt:
        zpieces.append((off, min(k_chunk, rt - off)))
        off += min(k_chunk, rt - off)

    mesh = plsc.VectorSubcoreMesh(core_axis_name="c", subcore_axis_name="s")

    assert n_chunks % 2 == 0
    scratch = [
        pltpu.VMEM_SHARED((n_nodes, DH), F32),
        pltpu.VMEM_SHARED((n_nodes, DH), F32),
        pltpu.VMEM((n_nodes,), F32),
    ]
    for _ in range(2):
        scratch += [
            pltpu.VMEM((k_chunk,), jnp.int32),
            pltpu.VMEM((k_chunk,), jnp.int32),
            pltpu.VMEM((k_chunk, DH), F32),
            pltpu.VMEM((k_chunk, DH), F32),
            pltpu.SemaphoreType.DMA,
            pltpu.SemaphoreType.DMA,
        ]

    @functools.partial(
        pl.kernel,
        out_type=(
            jax.ShapeDtypeStruct((n_nodes, D), F32),
            jax.ShapeDtypeStruct((NS, n_nodes), F32),
        ),
        mesh=mesh,
        scratch_types=scratch,
        compiler_params=pltpu.CompilerParams(use_tc_tiling_on_sc=False,
                                             needs_layout_passes=False),
    )
    def edge_agg(xh_hbm, msh_hbm, src_hbm, dst_hbm, out_acc, out_cnt,
                 table_sh, acc_sh, cnt,
                 sidx0, didx0, rows0, mbuf0, isem0, gsem0,
                 sidx1, didx1, rows1, mbuf1, isem1, gsem1):
        c = lax.axis_index("c")
        s = lax.axis_index("s")
        col0 = c * DH
        sidx = (sidx0, sidx1)
        didx = (didx0, didx1)
        rows = (rows0, rows1)
        mbuf = (mbuf0, mbuf1)
        isem = (isem0, isem1)
        gsem = (gsem0, gsem1)

        pltpu.sync_copy(
            xh_hbm.at[pl.ds(rt * s, rt), pl.ds(col0, DH)],
            table_sh.at[pl.ds(rt * s, rt)])

        zeros16 = jnp.zeros((16,), F32)

        def _zb(i, _):
            for j in range(DH // 16):
                rows0[i, pl.ds(j * 16, 16)] = zeros16
            return _
        lax.fori_loop(0, k_chunk, _zb, None)
        for zoff, zsz in zpieces:
            pltpu.sync_copy(rows0.at[pl.ds(0, zsz)],
                            acc_sh.at[pl.ds(rt * s + zoff, zsz)])

        def _zc(i, _):
            cnt[pl.ds(i * 16, 16)] = zeros16
            return _
        lax.fori_loop(0, n_nodes // 16, _zc, None)

        plsc.subcore_barrier()

        ones16 = jnp.ones((16,), F32)

        def _fire_in(k, p):
            base = s * et + k * k_chunk
            pltpu.async_copy(src_hbm.at[pl.ds(base, k_chunk)], sidx[p],
                             isem[p])
            pltpu.async_copy(dst_hbm.at[pl.ds(base, k_chunk)], didx[p],
                             isem[p])
            pltpu.async_copy(
                msh_hbm.at[pl.ds(base, k_chunk), pl.ds(col0, DH)], mbuf[p],
                isem[p])

        def _wait_in(k, p):
            base = s * et + k * k_chunk
            pltpu.make_async_copy(src_hbm.at[pl.ds(base, k_chunk)], sidx[p],
                                  isem[p]).wait()
            pltpu.make_async_copy(dst_hbm.at[pl.ds(base, k_chunk)], didx[p],
                                  isem[p]).wait()
            pltpu.make_async_copy(
                msh_hbm.at[pl.ds(base, k_chunk), pl.ds(col0, DH)], mbuf[p],
                isem[p]).wait()

        def _fire_gather(p):
            pltpu.async_copy(table_sh.at[sidx[p]], rows[p], gsem[p])

        def _wait_gather(p):
            pltpu.make_async_copy(table_sh.at[sidx[p]], rows[p],
                                  gsem[p]).wait()

        _fire_in(0, 0)
        _fire_in(1, 1)
        _wait_in(0, 0)
        _fire_gather(0)

        def _body(k, p):
            _wait_gather(p)

            @pl.when(k + 1 < n_chunks)
            def _():
                _wait_in(k + 1, p ^ 1)
                _fire_gather(p ^ 1)

            def _mul(i, _):
                for j in range(DH // 16):
                    sl = pl.ds(j * 16, 16)
                    rows[p][i, sl] = rows[p][i, sl] * mbuf[p][i, sl]
                return _
            lax.fori_loop(0, k_chunk, _mul, None)

            @pl.when(c == 0)
            def _():
                def _cnt(i, _):
                    idx = didx[p][pl.ds(i * 16, 16)]
                    plsc.addupdate_scatter(cnt, [idx], ones16)
                    return _
                lax.fori_loop(0, k_chunk // 16, _cnt, None)

            pltpu.sync_copy(rows[p], acc_sh.at[didx[p]], add=True)

            @pl.when(k + 2 < n_chunks)
            def _():
                _fire_in(k + 2, p)

        def _pair(kk, _):
            _body(kk * 2, 0)
            _body(kk * 2 + 1, 1)
            return _
        lax.fori_loop(0, n_chunks // 2, _pair, None)

        plsc.subcore_barrier()

        pltpu.sync_copy(
            acc_sh.at[pl.ds(rt * s, rt)],
            out_acc.at[pl.ds(rt * s, rt), pl.ds(col0, DH)])

        @pl.when(c == 0)
        def _():
            pltpu.sync_copy(cnt, out_cnt.at[s])

    return edge_agg


_edge_agg_atom = _make_edge_agg(N_ATOM, E_ATOM, 40)
_edge_agg_res = _make_edge_agg(N_RES_PAD, E_RES, 40)


def _dot(a, b):
    return jnp.dot(a, b, preferred_element_type=F32)


def _matmul_body(x_ref, w_ref, o_ref):
    o_ref[...] = _dot(x_ref[...], w_ref[...])


def _matmul(x, w):
    return pl.pallas_call(
        _matmul_body,
        out_shape=jax.ShapeDtypeStruct((x.shape[0], w.shape[1]), F32),
    )(x, w)


def _msh2_body(nb, ef_ref, sh_ref, Wb1_ref, bb1_ref, Wb2_ref, bb2_ref,
               Wr1_ref, br1_ref, Wr2_ref, br2_ref, Wsh_ref, o_ref):
    pid = pl.program_id(0)
    is_b = pid < nb
    W1 = jnp.where(is_b, Wb1_ref[...], Wr1_ref[...])
    b1 = jnp.where(is_b, bb1_ref[...], br1_ref[...])
    W2 = jnp.where(is_b, Wb2_ref[...], Wr2_ref[...])
    b2 = jnp.where(is_b, bb2_ref[...], br2_ref[...])
    h = jnp.maximum(_dot(ef_ref[...], W1) + b1, 0.0)
    w = _dot(h, W2) + b2
    o_ref[...] = _dot(sh_ref[...], Wsh_ref[...]) * w


def _msh_two_group(ef, sh, Wb1, bb1, Wb2, bb2, Wr1, br1, Wr2, br2, Wsh,
                   n_bond, blk):
    e = ef.shape[0]
    grid = e // blk
    nb = n_bond // blk
    full = lambda i: (0, 0)
    h = Wb1.shape[1]
    return pl.pallas_call(
        functools.partial(_msh2_body, nb),
        grid=(grid,),
        in_specs=[
            pl.BlockSpec((blk, 16), lambda i: (i, 0)),
            pl.BlockSpec((blk, 9), lambda i: (i, 0)),
            pl.BlockSpec((16, h), full), pl.BlockSpec((1, h), full),
            pl.BlockSpec((h, D), full), pl.BlockSpec((1, D), full),
            pl.BlockSpec((16, h), full), pl.BlockSpec((1, h), full),
            pl.BlockSpec((h, D), full), pl.BlockSpec((1, D), full),
            pl.BlockSpec((9, D), full),
        ],
        out_specs=pl.BlockSpec((blk, D), lambda i: (i, 0)),
        out_shape=jax.ShapeDtypeStruct((e, D), F32),
    )(ef, sh, Wb1, bb1.reshape(1, h), Wb2, bb2.reshape(1, D),
      Wr1, br1.reshape(1, h), Wr2, br2.reshape(1, D), Wsh)


def _msh1_body(ef_ref, sh_ref, W1_ref, b1_ref, W2_ref, b2_ref, Wsh_ref, o_ref):
    h = jnp.maximum(_dot(ef_ref[...], W1_ref[...]) + b1_ref[...], 0.0)
    w = _dot(h, W2_ref[...]) + b2_ref[...]
    o_ref[...] = _dot(sh_ref[...], Wsh_ref[...]) * w


def _msh_one_group(ef, sh, W1, b1, W2, b2, Wsh, blk):
    e = ef.shape[0]
    h = W1.shape[1]
    full = lambda i: (0, 0)
    return pl.pallas_call(
        _msh1_body,
        grid=(e // blk,),
        in_specs=[
            pl.BlockSpec((blk, 16), lambda i: (i, 0)),
            pl.BlockSpec((blk, 9), lambda i: (i, 0)),
            pl.BlockSpec((16, h), full), pl.BlockSpec((1, h), full),
            pl.BlockSpec((h, D), full), pl.BlockSpec((1, D), full),
            pl.BlockSpec((9, D), full),
        ],
        out_specs=pl.BlockSpec((blk, D), lambda i: (i, 0)),
        out_shape=jax.ShapeDtypeStruct((e, D), F32),
    )(ef, sh, W1, b1.reshape(1, h), W2, b2.reshape(1, D), Wsh)


def _seg_scale(cntp, ones_cols):
    tot = lax.dot_general(cntp, ones_cols, (((0,), (0,)), ((), ())),
                          preferred_element_type=F32)
    return 1.0 / jnp.maximum(tot, 1.0)


def _atom_post_body(acc_ref, cntp_ref, af_ref, aef_ref, ash_ref, Wout_ref,
                    Wa1_ref, ba1_ref, Wa2_ref, ba2_ref, Wxa_ref, Wsha_ref,
                    ao_ref, ma_ref):
    scale = _seg_scale(cntp_ref[0], jnp.ones((16, D), F32))
    agg = acc_ref[...] * scale
    atom_out = _dot(agg, Wout_ref[...]) + af_ref[...]
    ao_ref[...] = atom_out
    h = jnp.maximum(_dot(aef_ref[...], Wa1_ref[...]) + ba1_ref[...], 0.0)
    w = _dot(h, Wa2_ref[...]) + ba2_ref[...]
    ma_ref[...] = _dot(atom_out, Wxa_ref[...]) * \
        _dot(ash_ref[...], Wsha_ref[...]) * w


def _atom_post(acc, cntp, af, aef, ash, Wout, Wa1, ba1, Wa2, ba2, Wxa, Wsha,
               blk=2000):
    full = lambda i: (0, 0)
    h = Wa1.shape[1]
    return pl.pallas_call(
        _atom_post_body,
        grid=(N_ATOM // blk,),
        in_specs=[
            pl.BlockSpec((blk, D), lambda i: (i, 0)),
            pl.BlockSpec((1, 16, blk), lambda i: (i, 0, 0)),
            pl.BlockSpec((blk, D), lambda i: (i, 0)),
            pl.BlockSpec((blk, 16), lambda i: (i, 0)),
            pl.BlockSpec((blk, 9), lambda i: (i, 0)),
            pl.BlockSpec((D, D), full),
            pl.BlockSpec((16, h), full), pl.BlockSpec((1, h), full),
            pl.BlockSpec((h, D), full), pl.BlockSpec((1, D), full),
            pl.BlockSpec((D, D), full), pl.BlockSpec((9, D), full),
        ],
        out_specs=(pl.BlockSpec((blk, D), lambda i: (i, 0)),
                   pl.BlockSpec((blk, D), lambda i: (i, 0))),
        out_shape=(jax.ShapeDtypeStruct((N_ATOM, D), F32),
                   jax.ShapeDtypeStruct((N_ATOM, D), F32)),
    )(acc, cntp, af, aef, ash, Wout, Wa1, ba1.reshape(1, h), Wa2,
      ba2.reshape(1, D), Wxa, Wsha)


def _res_seg_body(ma_ref, batch_ref, sum_ref, cnt_ref):
    pid = pl.program_id(0)

    @pl.when(pid == 0)
    def _():
        sum_ref[...] = jnp.zeros_like(sum_ref)
        cnt_ref[...] = jnp.zeros_like(cnt_ref)

    ids = batch_ref[0]
    rows = lax.broadcasted_iota(jnp.int32, (N_RES_PAD, ids.shape[1]), 0)
    onehot = (rows == ids).astype(F32)
    sum_ref[...] += _dot(onehot, ma_ref[...])
    cnt_ref[...] += _dot(onehot, jnp.ones((ids.shape[1], D), F32))


def _res_seg(ma, batch, blk=2000):
    grid = N_ATOM // blk
    return pl.pallas_call(
        _res_seg_body,
        grid=(grid,),
        in_specs=[
            pl.BlockSpec((blk, D), lambda i: (i, 0)),
            pl.BlockSpec((1, 1, blk), lambda i: (i, 0, 0)),
        ],
        out_specs=(pl.BlockSpec((N_RES_PAD, D), lambda i: (0, 0)),
                   pl.BlockSpec((N_RES_PAD, D), lambda i: (0, 0))),
        out_shape=(jax.ShapeDtypeStruct((N_RES_PAD, D), F32),
                   jax.ShapeDtypeStruct((N_RES_PAD, D), F32)),
    )(ma, batch.reshape(grid, 1, blk))


def _res_mid_body(sum_ref, cnt_ref, rfp_ref, Wout_ref, Wx_ref,
                  mid_ref, xh_ref):
    mean = sum_ref[...] / jnp.maximum(cnt_ref[...], 1.0)
    mid_pad = _dot(mean, Wout_ref[...]) + rfp_ref[...]
    valid = lax.broadcasted_iota(jnp.int32, (N_RES_PAD, D), 0) < N_RES
    mid_pad = jnp.where(valid, mid_pad, 0.0)
    mid_ref[...] = mid_pad[:N_RES, :]
    xh_ref[...] = _dot(mid_pad, Wx_ref[...])


def _res_mid(rsum, rcnt, rf_pad, Wout, Wx):
    return pl.pallas_call(
        _res_mid_body,
        out_shape=(jax.ShapeDtypeStruct((N_RES, D), F32),
                   jax.ShapeDtypeStruct((N_RES_PAD, D), F32)),
    )(rsum, rcnt, rf_pad, Wout, Wx)


def _res_out_body(acc_ref, cntp_ref, mid_ref, Wout_ref, o_ref):
    scale = _seg_scale(cntp_ref[...], jnp.ones((16, D), F32))
    mean = acc_ref[...] * scale
    o_ref[...] = _dot(mean, Wout_ref[...])[:N_RES, :] + mid_ref[...]


def _res_out(acc, cntp, mid, Wout):
    return pl.pallas_call(
        _res_out_body,
        out_shape=jax.ShapeDtypeStruct((N_RES, D), F32),
    )(acc, cntp, mid, Wout)


def kernel(atom_features, atom_edge_index, bond_features, radius_edge_features,
           atom_edge_sh, res_features, atom_res_batch, agg_edge_features,
           agg_edge_sh, res_edge_index, res_edge_features, res_edge_sh,
           Wb1, bb1, Wb2, bb2, Wr1, br1, Wr2, br2, Wx_atom, Wsh_atom,
           Wout_atom, Wa1, ba1, Wa2, ba2, Wx_agg, Wsh_agg, Wout_agg,
           Wc1, bc1, Wc2, bc2, Wx_res, Wsh_res, Wout_res):
    src = atom_edge_index[0]
    dst = atom_edge_index[1]

    xh_atom = _matmul(atom_features, Wx_atom)
    ef_atom = jnp.concatenate([bond_features, radius_edge_features], axis=0)
    msh_atom = _msh_two_group(ef_atom, atom_edge_sh, Wb1, bb1, Wb2, bb2,
                              Wr1, br1, Wr2, br2, Wsh_atom, E_BOND, 2000)
    acc_a, cntp_a = _edge_agg_atom(xh_atom, msh_atom, src, dst)
    cntp_a = cntp_a.reshape(16, 5, 2000).transpose(1, 0, 2)

    atom_out, ma = _atom_post(acc_a, cntp_a, atom_features, agg_edge_features,
                              agg_edge_sh, Wout_atom, Wa1, ba1, Wa2, ba2,
                              Wx_agg, Wsh_agg)

    rsum, rcnt = _res_seg(ma, atom_res_batch)
    rf_pad = jnp.zeros((N_RES_PAD, D), F32).at[:N_RES].set(res_features)
    res_mid, xh_res = _res_mid(rsum, rcnt, rf_pad, Wout_agg, Wx_res)

    msh_res = _msh_one_group(res_edge_features, res_edge_sh,
                             Wc1, bc1, Wc2, bc2, Wsh_res, 2000)
    acc_r, cntp_r = _edge_agg_res(xh_res, msh_res,
                                  res_edge_index[0], res_edge_index[1])
    res_out = _res_out(acc_r, cntp_r, res_mid, Wout_res)

    return (atom_out, res_out)


# speedup vs baseline: 1.1734x; 1.1734x over previous
import functools

import jax
import jax.numpy as jnp
from jax import lax
from jax.experimental import pallas as pl
from jax.experimental.pallas import tpu as pltpu
from jax.experimental.pallas import tpu_sc as plsc

N_ATOM = 10000
N_RES = 1000
N_RES_PAD = 1024
E_BOND = 20000
E_ATOM = 160000
E_RES = 32000
D = 128

NC = 2
NS = 16
DH = D // NC

F32 = jnp.float32


def _make_edge_agg(n_nodes: int, n_edges: int, k_chunk: int):
    assert n_edges % (NS * k_chunk) == 0 and k_chunk % 8 == 0
    assert n_nodes % NS == 0
    et = n_edges // NS
    n_chunks = et // k_chunk
    rt = n_nodes // NS
    zpieces = []
    off = 0
    while off < rt:
        zpieces.append((off, min(k_chunk, rt - off)))
        off += min(k_chunk, rt - off)

    mesh = plsc.VectorSubcoreMesh(core_axis_name="c", subcore_axis_name="s")

    scratch = [
        pltpu.VMEM_SHARED((n_nodes, DH), F32),
        pltpu.VMEM_SHARED((n_nodes, DH), F32),
        pltpu.VMEM((n_nodes,), F32),
    ]
    for _ in range(2):
        scratch += [
            pltpu.VMEM((k_chunk,), jnp.int32),
            pltpu.VMEM((k_chunk,), jnp.int32),
            pltpu.VMEM((k_chunk, DH), F32),
            pltpu.VMEM((k_chunk, DH), F32),
            pltpu.SemaphoreType.DMA,
            pltpu.SemaphoreType.DMA,
        ]

    @functools.partial(
        pl.kernel,
        out_type=(
            jax.ShapeDtypeStruct((n_nodes, D), F32),
            jax.ShapeDtypeStruct((NS, n_nodes), F32),
        ),
        mesh=mesh,
        scratch_types=scratch,
        compiler_params=pltpu.CompilerParams(use_tc_tiling_on_sc=False,
                                             needs_layout_passes=False),
    )
    def edge_agg(xh_hbm, msh_hbm, src_hbm, dst_hbm, out_acc, out_cnt,
                 table_sh, acc_sh, cnt,
                 sidx0, didx0, rows0, mbuf0, isem0, gsem0,
                 sidx1, didx1, rows1, mbuf1, isem1, gsem1):
        c = lax.axis_index("c")
        s = lax.axis_index("s")
        col0 = c * DH
        sidx = (sidx0, sidx1)
        didx = (didx0, didx1)
        rows = (rows0, rows1)
        mbuf = (mbuf0, mbuf1)
        isem = (isem0, isem1)
        gsem = (gsem0, gsem1)

        pltpu.sync_copy(
            xh_hbm.at[pl.ds(rt * s, rt), pl.ds(col0, DH)],
            table_sh.at[pl.ds(rt * s, rt)])

        zeros16 = jnp.zeros((16,), F32)

        def _zb(i, _):
            for j in range(DH // 16):
                rows0[i, pl.ds(j * 16, 16)] = zeros16
            return _
        lax.fori_loop(0, k_chunk, _zb, None)
        for zoff, zsz in zpieces:
            pltpu.sync_copy(rows0.at[pl.ds(0, zsz)],
                            acc_sh.at[pl.ds(rt * s + zoff, zsz)])

        def _zc(i, _):
            cnt[pl.ds(i * 16, 16)] = zeros16
            return _
        lax.fori_loop(0, n_nodes // 16, _zc, None)

        plsc.subcore_barrier()

        ones16 = jnp.ones((16,), F32)

        def _fire_in(k, p):
            base = s * et + k * k_chunk
            pltpu.async_copy(src_hbm.at[pl.ds(base, k_chunk)], sidx[p],
                             isem[p])
            pltpu.async_copy(dst_hbm.at[pl.ds(base, k_chunk)], didx[p],
                             isem[p])
            pltpu.async_copy(
                msh_hbm.at[pl.ds(base, k_chunk), pl.ds(col0, DH)], mbuf[p],
                isem[p])

        def _wait_in(k, p):
            base = s * et + k * k_chunk
            pltpu.make_async_copy(src_hbm.at[pl.ds(base, k_chunk)], sidx[p],
                                  isem[p]).wait()
            pltpu.make_async_copy(dst_hbm.at[pl.ds(base, k_chunk)], didx[p],
                                  isem[p]).wait()
            pltpu.make_async_copy(
                msh_hbm.at[pl.ds(base, k_chunk), pl.ds(col0, DH)], mbuf[p],
                isem[p]).wait()

        def _fire_gather(p):
            pltpu.async_copy(table_sh.at[sidx[p]], rows[p], gsem[p])

        def _wait_gather(p):
            pltpu.make_async_copy(table_sh.at[sidx[p]], rows[p],
                                  gsem[p]).wait()

        _fire_in(0, 0)
        _fire_in(1, 1)
        _wait_in(0, 0)
        _fire_gather(0)

        def _body(k, p):
            _wait_gather(p)

            @pl.when(k + 1 < n_chunks)
            def _():
                _wait_in(k + 1, p ^ 1)
                _fire_gather(p ^ 1)

            @plsc.parallel_loop(0, k_chunk, unroll=8)
            def _mul(i):
                for j in range(DH // 16):
                    sl = pl.ds(j * 16, 16)
                    rows[p][i, sl] = rows[p][i, sl] * mbuf[p][i, sl]

            @pl.when(c == 0)
            def _():
                def _cnt(i, _):
                    idx = didx[p][pl.ds(i * 16, 16)]
                    plsc.addupdate_scatter(cnt, [idx], ones16)
                    return _
                lax.fori_loop(0, k_chunk // 16, _cnt, None)

            pltpu.sync_copy(rows[p], acc_sh.at[didx[p]], add=True)

            @pl.when(k + 2 < n_chunks)
            def _():
                _fire_in(k + 2, p)

        def _pair(kk, _):
            _body(kk * 2, 0)
            _body(kk * 2 + 1, 1)
            return _
        lax.fori_loop(0, n_chunks // 2, _pair, None)
        if n_chunks % 2:
            _body(jnp.int32(n_chunks - 1), 0)

        plsc.subcore_barrier()

        pltpu.sync_copy(
            acc_sh.at[pl.ds(rt * s, rt)],
            out_acc.at[pl.ds(rt * s, rt), pl.ds(col0, DH)])

        @pl.when(c == 0)
        def _():
            pltpu.sync_copy(cnt, out_cnt.at[s])

    return edge_agg


_edge_agg_atom = _make_edge_agg(N_ATOM, E_ATOM, 80)
_edge_agg_res = _make_edge_agg(N_RES_PAD, E_RES, 80)


def _dot(a, b):
    return jnp.dot(a, b, preferred_element_type=F32)


def _matmul_body(x_ref, w_ref, o_ref):
    o_ref[...] = _dot(x_ref[...], w_ref[...])


def _matmul(x, w):
    return pl.pallas_call(
        _matmul_body,
        out_shape=jax.ShapeDtypeStruct((x.shape[0], w.shape[1]), F32),
    )(x, w)


def _msh2_body(nb, ef_ref, sh_ref, Wb1_ref, bb1_ref, Wb2_ref, bb2_ref,
               Wr1_ref, br1_ref, Wr2_ref, br2_ref, Wsh_ref, o_ref):
    pid = pl.program_id(0)
    is_b = pid < nb
    W1 = jnp.where(is_b, Wb1_ref[...], Wr1_ref[...])
    b1 = jnp.where(is_b, bb1_ref[...], br1_ref[...])
    W2 = jnp.where(is_b, Wb2_ref[...], Wr2_ref[...])
    b2 = jnp.where(is_b, bb2_ref[...], br2_ref[...])
    h = jnp.maximum(_dot(ef_ref[...], W1) + b1, 0.0)
    w = _dot(h, W2) + b2
    o_ref[...] = _dot(sh_ref[...], Wsh_ref[...]) * w


def _msh_two_group(ef, sh, Wb1, bb1, Wb2, bb2, Wr1, br1, Wr2, br2, Wsh,
                   n_bond, blk):
    e = ef.shape[0]
    grid = e // blk
    nb = n_bond // blk
    full = lambda i: (0, 0)
    h = Wb1.shape[1]
    return pl.pallas_call(
        functools.partial(_msh2_body, nb),
        grid=(grid,),
        in_specs=[
            pl.BlockSpec((blk, 16), lambda i: (i, 0)),
            pl.BlockSpec((blk, 9), lambda i: (i, 0)),
            pl.BlockSpec((16, h), full), pl.BlockSpec((1, h), full),
            pl.BlockSpec((h, D), full), pl.BlockSpec((1, D), full),
            pl.BlockSpec((16, h), full), pl.BlockSpec((1, h), full),
            pl.BlockSpec((h, D), full), pl.BlockSpec((1, D), full),
            pl.BlockSpec((9, D), full),
        ],
        out_specs=pl.BlockSpec((blk, D), lambda i: (i, 0)),
        out_shape=jax.ShapeDtypeStruct((e, D), F32),
    )(ef, sh, Wb1, bb1.reshape(1, h), Wb2, bb2.reshape(1, D),
      Wr1, br1.reshape(1, h), Wr2, br2.reshape(1, D), Wsh)


def _msh1_body(ef_ref, sh_ref, W1_ref, b1_ref, W2_ref, b2_ref, Wsh_ref, o_ref):
    h = jnp.maximum(_dot(ef_ref[...], W1_ref[...]) + b1_ref[...], 0.0)
    w = _dot(h, W2_ref[...]) + b2_ref[...]
    o_ref[...] = _dot(sh_ref[...], Wsh_ref[...]) * w


def _msh_one_group(ef, sh, W1, b1, W2, b2, Wsh, blk):
    e = ef.shape[0]
    h = W1.shape[1]
    full = lambda i: (0, 0)
    return pl.pallas_call(
        _msh1_body,
        grid=(e // blk,),
        in_specs=[
            pl.BlockSpec((blk, 16), lambda i: (i, 0)),
            pl.BlockSpec((blk, 9), lambda i: (i, 0)),
            pl.BlockSpec((16, h), full), pl.BlockSpec((1, h), full),
            pl.BlockSpec((h, D), full), pl.BlockSpec((1, D), full),
            pl.BlockSpec((9, D), full),
        ],
        out_specs=pl.BlockSpec((blk, D), lambda i: (i, 0)),
        out_shape=jax.ShapeDtypeStruct((e, D), F32),
    )(ef, sh, W1, b1.reshape(1, h), W2, b2.reshape(1, D), Wsh)


def _seg_scale(cntp, ones_cols):
    tot = lax.dot_general(cntp, ones_cols, (((0,), (0,)), ((), ())),
                          preferred_element_type=F32)
    return 1.0 / jnp.maximum(tot, 1.0)


def _atom_post_body(acc_ref, cntp_ref, af_ref, aef_ref, ash_ref, Wout_ref,
                    Wa1_ref, ba1_ref, Wa2_ref, ba2_ref, Wxa_ref, Wsha_ref,
                    ao_ref, ma_ref):
    scale = _seg_scale(cntp_ref[0], jnp.ones((16, D), F32))
    agg = acc_ref[...] * scale
    atom_out = _dot(agg, Wout_ref[...]) + af_ref[...]
    ao_ref[...] = atom_out
    h = jnp.maximum(_dot(aef_ref[...], Wa1_ref[...]) + ba1_ref[...], 0.0)
    w = _dot(h, Wa2_ref[...]) + ba2_ref[...]
    ma_ref[...] = _dot(atom_out, Wxa_ref[...]) * \
        _dot(ash_ref[...], Wsha_ref[...]) * w


def _atom_post(acc, cntp, af, aef, ash, Wout, Wa1, ba1, Wa2, ba2, Wxa, Wsha,
               blk=2000):
    full = lambda i: (0, 0)
    h = Wa1.shape[1]
    return pl.pallas_call(
        _atom_post_body,
        grid=(N_ATOM // blk,),
        in_specs=[
            pl.BlockSpec((blk, D), lambda i: (i, 0)),
            pl.BlockSpec((1, 16, blk), lambda i: (i, 0, 0)),
            pl.BlockSpec((blk, D), lambda i: (i, 0)),
            pl.BlockSpec((blk, 16), lambda i: (i, 0)),
            pl.BlockSpec((blk, 9), lambda i: (i, 0)),
            pl.BlockSpec((D, D), full),
            pl.BlockSpec((16, h), full), pl.BlockSpec((1, h), full),
            pl.BlockSpec((h, D), full), pl.BlockSpec((1, D), full),
            pl.BlockSpec((D, D), full), pl.BlockSpec((9, D), full),
        ],
        out_specs=(pl.BlockSpec((blk, D), lambda i: (i, 0)),
                   pl.BlockSpec((blk, D), lambda i: (i, 0))),
        out_shape=(jax.ShapeDtypeStruct((N_ATOM, D), F32),
                   jax.ShapeDtypeStruct((N_ATOM, D), F32)),
    )(acc, cntp, af, aef, ash, Wout, Wa1, ba1.reshape(1, h), Wa2,
      ba2.reshape(1, D), Wxa, Wsha)


def _res_seg_body(ma_ref, batch_ref, sum_ref, cnt_ref):
    pid = pl.program_id(0)

    @pl.when(pid == 0)
    def _():
        sum_ref[...] = jnp.zeros_like(sum_ref)
        cnt_ref[...] = jnp.zeros_like(cnt_ref)

    ids = batch_ref[0]
    rows = lax.broadcasted_iota(jnp.int32, (N_RES_PAD, ids.shape[1]), 0)
    onehot = (rows == ids).astype(F32)
    sum_ref[...] += _dot(onehot, ma_ref[...])
    cnt_ref[...] += _dot(onehot, jnp.ones((ids.shape[1], D), F32))


def _res_seg(ma, batch, blk=2000):
    grid = N_ATOM // blk
    return pl.pallas_call(
        _res_seg_body,
        grid=(grid,),
        in_specs=[
            pl.BlockSpec((blk, D), lambda i: (i, 0)),
            pl.BlockSpec((1, 1, blk), lambda i: (i, 0, 0)),
        ],
        out_specs=(pl.BlockSpec((N_RES_PAD, D), lambda i: (0, 0)),
                   pl.BlockSpec((N_RES_PAD, D), lambda i: (0, 0))),
        out_shape=(jax.ShapeDtypeStruct((N_RES_PAD, D), F32),
                   jax.ShapeDtypeStruct((N_RES_PAD, D), F32)),
    )(ma, batch.reshape(grid, 1, blk))


def _res_mid_body(sum_ref, cnt_ref, rfp_ref, Wout_ref, Wx_ref,
                  mid_ref, xh_ref):
    mean = sum_ref[...] / jnp.maximum(cnt_ref[...], 1.0)
    mid_pad = _dot(mean, Wout_ref[...]) + rfp_ref[...]
    valid = lax.broadcasted_iota(jnp.int32, (N_RES_PAD, D), 0) < N_RES
    mid_pad = jnp.where(valid, mid_pad, 0.0)
    mid_ref[...] = mid_pad[:N_RES, :]
    xh_ref[...] = _dot(mid_pad, Wx_ref[...])


def _res_mid(rsum, rcnt, rf_pad, Wout, Wx):
    return pl.pallas_call(
        _res_mid_body,
        out_shape=(jax.ShapeDtypeStruct((N_RES, D), F32),
                   jax.ShapeDtypeStruct((N_RES_PAD, D), F32)),
    )(rsum, rcnt, rf_pad, Wout, Wx)


def _res_out_body(acc_ref, cntp_ref, mid_ref, Wout_ref, o_ref):
    scale = _seg_scale(cntp_ref[...], jnp.ones((16, D), F32))
    mean = acc_ref[...] * scale
    o_ref[...] = _dot(mean, Wout_ref[...])[:N_RES, :] + mid_ref[...]


def _res_out(acc, cntp, mid, Wout):
    return pl.pallas_call(
        _res_out_body,
        out_shape=jax.ShapeDtypeStruct((N_RES, D), F32),
    )(acc, cntp, mid, Wout)


def kernel(atom_features, atom_edge_index, bond_features, radius_edge_features,
           atom_edge_sh, res_features, atom_res_batch, agg_edge_features,
           agg_edge_sh, res_edge_index, res_edge_features, res_edge_sh,
           Wb1, bb1, Wb2, bb2, Wr1, br1, Wr2, br2, Wx_atom, Wsh_atom,
           Wout_atom, Wa1, ba1, Wa2, ba2, Wx_agg, Wsh_agg, Wout_agg,
           Wc1, bc1, Wc2, bc2, Wx_res, Wsh_res, Wout_res):
    src = atom_edge_index[0]
    dst = atom_edge_index[1]

    xh_atom = _matmul(atom_features, Wx_atom)
    ef_atom = jnp.concatenate([bond_features, radius_edge_features], axis=0)
    msh_atom = _msh_two_group(ef_atom, atom_edge_sh, Wb1, bb1, Wb2, bb2,
                              Wr1, br1, Wr2, br2, Wsh_atom, E_BOND, 2000)
    acc_a, cntp_a = _edge_agg_atom(xh_atom, msh_atom, src, dst)
    cntp_a = cntp_a.reshape(16, 5, 2000).transpose(1, 0, 2)

    atom_out, ma = _atom_post(acc_a, cntp_a, atom_features, agg_edge_features,
                              agg_edge_sh, Wout_atom, Wa1, ba1, Wa2, ba2,
                              Wx_agg, Wsh_agg)

    rsum, rcnt = _res_seg(ma, atom_res_batch)
    rf_pad = jnp.zeros((N_RES_PAD, D), F32).at[:N_RES].set(res_features)
    res_mid, xh_res = _res_mid(rsum, rcnt, rf_pad, Wout_agg, Wx_res)

    msh_res = _msh_one_group(res_edge_features, res_edge_sh,
                             Wc1, bc1, Wc2, bc2, Wsh_res, 2000)
    acc_r, cntp_r = _edge_agg_res(xh_res, msh_res,
                                  res_edge_index[0], res_edge_index[1])
    res_out = _res_out(acc_r, cntp_r, res_mid, Wout_res)

    return (atom_out, res_out)


# speedup vs baseline: 1.3465x; 1.1475x over previous
import functools

import jax
import jax.numpy as jnp
from jax import lax
from jax.experimental import pallas as pl
from jax.experimental.pallas import tpu as pltpu
from jax.experimental.pallas import tpu_sc as plsc

N_ATOM = 10000
N_RES = 1000
N_RES_PAD = 1024
E_BOND = 20000
E_ATOM = 160000
E_RES = 32000
D = 128

NC = 2
NS = 16
DH = D // NC

F32 = jnp.float32


def _make_edge_agg(n_nodes: int, n_edges: int, k_chunk: int):
    assert n_edges % (NS * k_chunk) == 0 and k_chunk % 8 == 0
    assert n_nodes % NS == 0
    et = n_edges // NS
    n_chunks = et // k_chunk
    rt = n_nodes // NS
    zpieces = []
    off = 0
    while off < rt:
        zpieces.append((off, min(k_chunk, rt - off)))
        off += min(k_chunk, rt - off)

    mesh = plsc.VectorSubcoreMesh(core_axis_name="c", subcore_axis_name="s")

    scratch = [
        pltpu.VMEM_SHARED((n_nodes, DH), F32),
        pltpu.VMEM_SHARED((n_nodes, DH), F32),
        pltpu.VMEM((n_nodes,), F32),
    ]
    for _ in range(3):
        scratch += [
            pltpu.VMEM((k_chunk,), jnp.int32),
            pltpu.VMEM((k_chunk,), jnp.int32),
            pltpu.VMEM((k_chunk, DH), F32),
            pltpu.VMEM((k_chunk, DH), F32),
            pltpu.SemaphoreType.DMA,
            pltpu.SemaphoreType.DMA,
            pltpu.SemaphoreType.DMA,
        ]

    @functools.partial(
        pl.kernel,
        out_type=(
            jax.ShapeDtypeStruct((n_nodes, D), F32),
            jax.ShapeDtypeStruct((NS, n_nodes), F32),
        ),
        mesh=mesh,
        scratch_types=scratch,
        compiler_params=pltpu.CompilerParams(use_tc_tiling_on_sc=False,
                                             needs_layout_passes=False),
    )
    def edge_agg(xh_hbm, msh_hbm, src_hbm, dst_hbm, out_acc, out_cnt,
                 table_sh, acc_sh, cnt,
                 sidx0, didx0, rows0, mbuf0, isem0, gsem0, ssem0,
                 sidx1, didx1, rows1, mbuf1, isem1, gsem1, ssem1,
                 sidx2, didx2, rows2, mbuf2, isem2, gsem2, ssem2):
        c = lax.axis_index("c")
        s = lax.axis_index("s")
        col0 = c * DH
        sidx = (sidx0, sidx1, sidx2)
        didx = (didx0, didx1, didx2)
        rows = (rows0, rows1, rows2)
        mbuf = (mbuf0, mbuf1, mbuf2)
        isem = (isem0, isem1, isem2)
        gsem = (gsem0, gsem1, gsem2)
        ssem = (ssem0, ssem1, ssem2)

        pltpu.sync_copy(
            xh_hbm.at[pl.ds(rt * s, rt), pl.ds(col0, DH)],
            table_sh.at[pl.ds(rt * s, rt)])

        zeros16 = jnp.zeros((16,), F32)

        def _zb(i, _):
            for j in range(DH // 16):
                rows0[i, pl.ds(j * 16, 16)] = zeros16
            return _
        lax.fori_loop(0, k_chunk, _zb, None)
        for zoff, zsz in zpieces:
            pltpu.sync_copy(rows0.at[pl.ds(0, zsz)],
                            acc_sh.at[pl.ds(rt * s + zoff, zsz)])

        def _zc(i, _):
            cnt[pl.ds(i * 16, 16)] = zeros16
            return _
        lax.fori_loop(0, n_nodes // 16, _zc, None)

        plsc.subcore_barrier()

        ones16 = jnp.ones((16,), F32)

        def _fire_in(k, p):
            base = s * et + k * k_chunk
            pltpu.async_copy(src_hbm.at[pl.ds(base, k_chunk)], sidx[p],
                             isem[p])
            pltpu.async_copy(dst_hbm.at[pl.ds(base, k_chunk)], didx[p],
                             isem[p])
            pltpu.async_copy(
                msh_hbm.at[pl.ds(base, k_chunk), pl.ds(col0, DH)], mbuf[p],
                isem[p])

        def _wait_in(k, p):
            base = s * et + k * k_chunk
            pltpu.make_async_copy(src_hbm.at[pl.ds(base, k_chunk)], sidx[p],
                                  isem[p]).wait()
            pltpu.make_async_copy(dst_hbm.at[pl.ds(base, k_chunk)], didx[p],
                                  isem[p]).wait()
            pltpu.make_async_copy(
                msh_hbm.at[pl.ds(base, k_chunk), pl.ds(col0, DH)], mbuf[p],
                isem[p]).wait()

        def _fire_gather(p):
            pltpu.async_copy(table_sh.at[sidx[p]], rows[p], gsem[p])

        def _wait_gather(p):
            pltpu.make_async_copy(table_sh.at[sidx[p]], rows[p],
                                  gsem[p]).wait()

        def _fire_scatter(p):
            pltpu.async_copy(rows[p], acc_sh.at[didx[p]], ssem[p], add=True)

        def _wait_scatter(p):
            pltpu.make_async_copy(rows[p], acc_sh.at[didx[p]],
                                  ssem[p]).wait()

        _fire_in(0, 0)
        _fire_in(1, 1)
        _wait_in(0, 0)
        _fire_gather(0)

        def _body(k, p):
            pn = (p + 1) % 3
            pp = (p + 2) % 3
            _wait_gather(p)

            @pl.when(k + 1 < n_chunks)
            def _():
                _wait_in(k + 1, pn)
                _fire_gather(pn)

            @plsc.parallel_loop(0, k_chunk, unroll=8)
            def _mul(i):
                for j in range(DH // 16):
                    sl = pl.ds(j * 16, 16)
                    rows[p][i, sl] = rows[p][i, sl] * mbuf[p][i, sl]

            @pl.when(c == 0)
            def _():
                def _cnt(i, _):
                    idx = didx[p][pl.ds(i * 16, 16)]
                    plsc.addupdate_scatter(cnt, [idx], ones16)
                    return _
                lax.fori_loop(0, k_chunk // 16, _cnt, None)

            @pl.when(k >= 1)
            def _():
                _wait_scatter(pp)

            @pl.when(k + 2 < n_chunks)
            def _():
                _fire_in(k + 2, pp)

            _fire_scatter(p)

        def _triple(kk, _):
            _body(kk * 3, 0)
            _body(kk * 3 + 1, 1)
            _body(kk * 3 + 2, 2)
            return _
        lax.fori_loop(0, n_chunks // 3, _triple, None)
        for k in range(n_chunks - n_chunks % 3, n_chunks):
            _body(jnp.int32(k), k % 3)
        _wait_scatter((n_chunks - 1) % 3)

        plsc.subcore_barrier()

        pltpu.sync_copy(
            acc_sh.at[pl.ds(rt * s, rt)],
            out_acc.at[pl.ds(rt * s, rt), pl.ds(col0, DH)])

        @pl.when(c == 0)
        def _():
            pltpu.sync_copy(cnt, out_cnt.at[s])

    return edge_agg


_edge_agg_atom = _make_edge_agg(N_ATOM, E_ATOM, 80)
_edge_agg_res = _make_edge_agg(N_RES_PAD, E_RES, 80)


def _dot(a, b):
    return jnp.dot(a, b, preferred_element_type=F32)


def _matmul_body(x_ref, w_ref, o_ref):
    o_ref[...] = _dot(x_ref[...], w_ref[...])


def _matmul(x, w):
    return pl.pallas_call(
        _matmul_body,
        out_shape=jax.ShapeDtypeStruct((x.shape[0], w.shape[1]), F32),
    )(x, w)


def _msh2_body(nb, ef_ref, sh_ref, Wb1_ref, bb1_ref, Wb2_ref, bb2_ref,
               Wr1_ref, br1_ref, Wr2_ref, br2_ref, Wsh_ref, o_ref):
    pid = pl.program_id(0)
    is_b = pid < nb
    W1 = jnp.where(is_b, Wb1_ref[...], Wr1_ref[...])
    b1 = jnp.where(is_b, bb1_ref[...], br1_ref[...])
    W2 = jnp.where(is_b, Wb2_ref[...], Wr2_ref[...])
    b2 = jnp.where(is_b, bb2_ref[...], br2_ref[...])
    h = jnp.maximum(_dot(ef_ref[...], W1) + b1, 0.0)
    w = _dot(h, W2) + b2
    o_ref[...] = _dot(sh_ref[...], Wsh_ref[...]) * w


def _msh_two_group(ef, sh, Wb1, bb1, Wb2, bb2, Wr1, br1, Wr2, br2, Wsh,
                   n_bond, blk):
    e = ef.shape[0]
    grid = e // blk
    nb = n_bond // blk
    full = lambda i: (0, 0)
    h = Wb1.shape[1]
    return pl.pallas_call(
        functools.partial(_msh2_body, nb),
        grid=(grid,),
        in_specs=[
            pl.BlockSpec((blk, 16), lambda i: (i, 0)),
            pl.BlockSpec((blk, 9), lambda i: (i, 0)),
            pl.BlockSpec((16, h), full), pl.BlockSpec((1, h), full),
            pl.BlockSpec((h, D), full), pl.BlockSpec((1, D), full),
            pl.BlockSpec((16, h), full), pl.BlockSpec((1, h), full),
            pl.BlockSpec((h, D), full), pl.BlockSpec((1, D), full),
            pl.BlockSpec((9, D), full),
        ],
        out_specs=pl.BlockSpec((blk, D), lambda i: (i, 0)),
        out_shape=jax.ShapeDtypeStruct((e, D), F32),
    )(ef, sh, Wb1, bb1.reshape(1, h), Wb2, bb2.reshape(1, D),
      Wr1, br1.reshape(1, h), Wr2, br2.reshape(1, D), Wsh)


def _msh1_body(ef_ref, sh_ref, W1_ref, b1_ref, W2_ref, b2_ref, Wsh_ref, o_ref):
    h = jnp.maximum(_dot(ef_ref[...], W1_ref[...]) + b1_ref[...], 0.0)
    w = _dot(h, W2_ref[...]) + b2_ref[...]
    o_ref[...] = _dot(sh_ref[...], Wsh_ref[...]) * w


def _msh_one_group(ef, sh, W1, b1, W2, b2, Wsh, blk):
    e = ef.shape[0]
    h = W1.shape[1]
    full = lambda i: (0, 0)
    return pl.pallas_call(
        _msh1_body,
        grid=(e // blk,),
        in_specs=[
            pl.BlockSpec((blk, 16), lambda i: (i, 0)),
            pl.BlockSpec((blk, 9), lambda i: (i, 0)),
            pl.BlockSpec((16, h), full), pl.BlockSpec((1, h), full),
            pl.BlockSpec((h, D), full), pl.BlockSpec((1, D), full),
            pl.BlockSpec((9, D), full),
        ],
        out_specs=pl.BlockSpec((blk, D), lambda i: (i, 0)),
        out_shape=jax.ShapeDtypeStruct((e, D), F32),
    )(ef, sh, W1, b1.reshape(1, h), W2, b2.reshape(1, D), Wsh)


def _seg_scale(cntp, ones_cols):
    tot = lax.dot_general(cntp, ones_cols, (((0,), (0,)), ((), ())),
                          preferred_element_type=F32)
    return 1.0 / jnp.maximum(tot, 1.0)


def _atom_post_body(acc_ref, cntp_ref, af_ref, aef_ref, ash_ref, Wout_ref,
                    Wa1_ref, ba1_ref, Wa2_ref, ba2_ref, Wxa_ref, Wsha_ref,
                    ao_ref, ma_ref):
    scale = _seg_scale(cntp_ref[0], jnp.ones((16, D), F32))
    agg = acc_ref[...] * scale
    atom_out = _dot(agg, Wout_ref[...]) + af_ref[...]
    ao_ref[...] = atom_out
    h = jnp.maximum(_dot(aef_ref[...], Wa1_ref[...]) + ba1_ref[...], 0.0)
    w = _dot(h, Wa2_ref[...]) + ba2_ref[...]
    ma_ref[...] = _dot(atom_out, Wxa_ref[...]) * \
        _dot(ash_ref[...], Wsha_ref[...]) * w


def _atom_post(acc, cntp, af, aef, ash, Wout, Wa1, ba1, Wa2, ba2, Wxa, Wsha,
               blk=2000):
    full = lambda i: (0, 0)
    h = Wa1.shape[1]
    return pl.pallas_call(
        _atom_post_body,
        grid=(N_ATOM // blk,),
        in_specs=[
            pl.BlockSpec((blk, D), lambda i: (i, 0)),
            pl.BlockSpec((1, 16, blk), lambda i: (i, 0, 0)),
            pl.BlockSpec((blk, D), lambda i: (i, 0)),
            pl.BlockSpec((blk, 16), lambda i: (i, 0)),
            pl.BlockSpec((blk, 9), lambda i: (i, 0)),
            pl.BlockSpec((D, D), full),
            pl.BlockSpec((16, h), full), pl.BlockSpec((1, h), full),
            pl.BlockSpec((h, D), full), pl.BlockSpec((1, D), full),
            pl.BlockSpec((D, D), full), pl.BlockSpec((9, D), full),
        ],
        out_specs=(pl.BlockSpec((blk, D), lambda i: (i, 0)),
                   pl.BlockSpec((blk, D), lambda i: (i, 0))),
        out_shape=(jax.ShapeDtypeStruct((N_ATOM, D), F32),
                   jax.ShapeDtypeStruct((N_ATOM, D), F32)),
    )(acc, cntp, af, aef, ash, Wout, Wa1, ba1.reshape(1, h), Wa2,
      ba2.reshape(1, D), Wxa, Wsha)


def _res_seg_body(ma_ref, batch_ref, sum_ref, cnt_ref):
    pid = pl.program_id(0)

    @pl.when(pid == 0)
    def _():
        sum_ref[...] = jnp.zeros_like(sum_ref)
        cnt_ref[...] = jnp.zeros_like(cnt_ref)

    ids = batch_ref[0]
    rows = lax.broadcasted_iota(jnp.int32, (N_RES_PAD, ids.shape[1]), 0)
    onehot = (rows == ids).astype(F32)
    sum_ref[...] += _dot(onehot, ma_ref[...])
    cnt_ref[...] += _dot(onehot, jnp.ones((ids.shape[1], D), F32))


def _res_seg(ma, batch, blk=2000):
    grid = N_ATOM // blk
    return pl.pallas_call(
        _res_seg_body,
        grid=(grid,),
        in_specs=[
            pl.BlockSpec((blk, D), lambda i: (i, 0)),
            pl.BlockSpec((1, 1, blk), lambda i: (i, 0, 0)),
        ],
        out_specs=(pl.BlockSpec((N_RES_PAD, D), lambda i: (0, 0)),
                   pl.BlockSpec((N_RES_PAD, D), lambda i: (0, 0))),
        out_shape=(jax.ShapeDtypeStruct((N_RES_PAD, D), F32),
                   jax.ShapeDtypeStruct((N_RES_PAD, D), F32)),
    )(ma, batch.reshape(grid, 1, blk))


def _res_mid_body(sum_ref, cnt_ref, rfp_ref, Wout_ref, Wx_ref,
                  mid_ref, xh_ref):
    mean = sum_ref[...] / jnp.maximum(cnt_ref[...], 1.0)
    mid_pad = _dot(mean, Wout_ref[...]) + rfp_ref[...]
    valid = lax.broadcasted_iota(jnp.int32, (N_RES_PAD, D), 0) < N_RES
    mid_pad = jnp.where(valid, mid_pad, 0.0)
    mid_ref[...] = mid_pad[:N_RES, :]
    xh_ref[...] = _dot(mid_pad, Wx_ref[...])


def _res_mid(rsum, rcnt, rf_pad, Wout, Wx):
    return pl.pallas_call(
        _res_mid_body,
        out_shape=(jax.ShapeDtypeStruct((N_RES, D), F32),
                   jax.ShapeDtypeStruct((N_RES_PAD, D), F32)),
    )(rsum, rcnt, rf_pad, Wout, Wx)


def _res_out_body(acc_ref, cntp_ref, mid_ref, Wout_ref, o_ref):
    scale = _seg_scale(cntp_ref[...], jnp.ones((16, D), F32))
    mean = acc_ref[...] * scale
    o_ref[...] = _dot(mean, Wout_ref[...])[:N_RES, :] + mid_ref[...]


def _res_out(acc, cntp, mid, Wout):
    return pl.pallas_call(
        _res_out_body,
        out_shape=jax.ShapeDtypeStruct((N_RES, D), F32),
    )(acc, cntp, mid, Wout)


def kernel(atom_features, atom_edge_index, bond_features, radius_edge_features,
           atom_edge_sh, res_features, atom_res_batch, agg_edge_features,
           agg_edge_sh, res_edge_index, res_edge_features, res_edge_sh,
           Wb1, bb1, Wb2, bb2, Wr1, br1, Wr2, br2, Wx_atom, Wsh_atom,
           Wout_atom, Wa1, ba1, Wa2, ba2, Wx_agg, Wsh_agg, Wout_agg,
           Wc1, bc1, Wc2, bc2, Wx_res, Wsh_res, Wout_res):
    src = atom_edge_index[0]
    dst = atom_edge_index[1]

    xh_atom = _matmul(atom_features, Wx_atom)
    ef_atom = jnp.concatenate([bond_features, radius_edge_features], axis=0)
    msh_atom = _msh_two_group(ef_atom, atom_edge_sh, Wb1, bb1, Wb2, bb2,
                              Wr1, br1, Wr2, br2, Wsh_atom, E_BOND, 2000)
    acc_a, cntp_a = _edge_agg_atom(xh_atom, msh_atom, src, dst)
    cntp_a = cntp_a.reshape(16, 5, 2000).transpose(1, 0, 2)

    atom_out, ma = _atom_post(acc_a, cntp_a, atom_features, agg_edge_features,
                              agg_edge_sh, Wout_atom, Wa1, ba1, Wa2, ba2,
                              Wx_agg, Wsh_agg)

    rsum, rcnt = _res_seg(ma, atom_res_batch)
    rf_pad = jnp.zeros((N_RES_PAD, D), F32).at[:N_RES].set(res_features)
    res_mid, xh_res = _res_mid(rsum, rcnt, rf_pad, Wout_agg, Wx_res)

    msh_res = _msh_one_group(res_edge_features, res_edge_sh,
                             Wc1, bc1, Wc2, bc2, Wsh_res, 2000)
    acc_r, cntp_r = _edge_agg_res(xh_res, msh_res,
                                  res_edge_index[0], res_edge_index[1])
    res_out = _res_out(acc_r, cntp_r, res_mid, Wout_res)

    return (atom_out, res_out)
